# baseline (speedup 1.0000x reference)
import functools
import math

import jax
import jax.numpy as jnp
from jax import lax
from jax.experimental import pallas as pl
from jax.experimental.pallas import tpu as pltpu
from jax.experimental.pallas import tpu_sc as plsc

_FP = 32
_NATOM = 10000
_NBOND = 160000
_NMOL = 512
_K = 3
_T = 3
_S = 1.0 / math.sqrt(1.0 + 1e-6)
_EBLK = 2000
_NEB = _NBOND // _EBLK


def _leaky(x):
    return jnp.where(x >= 0, x, 0.01 * x)


def _sigmoid(x):
    return 1.0 / (1.0 + jnp.exp(-x))


def _elu(x):
    return jnp.where(x > 0, x, jnp.exp(jnp.minimum(x, 0.0)) - 1.0)


_NW = 32
_GB = _NBOND * 2
_GPW = _GB // _NW
_GCH = 400
_GNC = _GPW // _GCH
_GD = 128


def _sc_gather_body(table, idx, out, idx_v, rows_v, sem):
    wid = lax.axis_index("s") * 2 + lax.axis_index("c")
    base = wid * _GPW
    for c in range(_GNC):
        off = base + c * _GCH
        pltpu.sync_copy(idx.at[pl.ds(off, _GCH)], idx_v)
        pltpu.async_copy(table.at[idx_v], rows_v, sem).wait()
        pltpu.sync_copy(rows_v, out.at[pl.ds(off, _GCH)])


_sc_gather = pl.kernel(
    _sc_gather_body,
    mesh=plsc.VectorSubcoreMesh(core_axis_name="c", subcore_axis_name="s"),
    out_type=jax.ShapeDtypeStruct((_GB, _GD), jnp.float32),
    scratch_types=[
        pltpu.VMEM((_GCH,), jnp.int32),
        pltpu.VMEM((_GCH, _GD), jnp.float32),
        pltpu.SemaphoreType.DMA,
    ],
)


def _edge_body(nbrx_ref, srcx_ref, bond_ref, encw_ref, encb_ref, rmat_ref,
               smat_ref, attw_ref, attb_ref, alwt_ref, alwn_ref, alb_ref,
               att_ref, score_ref):
    benc = jnp.dot(bond_ref[...], encw_ref[...],
                   preferred_element_type=jnp.float32) + encb_ref[...]
    benc = jnp.maximum(benc, 0.0)
    ah = jnp.dot(nbrx_ref[...], rmat_ref[...],
                 preferred_element_type=jnp.float32)
    neighbor = jnp.dot(ah * benc, smat_ref[...],
                       preferred_element_type=jnp.float32)
    att_ref[...] = jnp.dot(neighbor, attw_ref[...],
                           preferred_element_type=jnp.float32) + attb_ref[...]
    sc = (jnp.dot(srcx_ref[...], alwt_ref[...],
                  preferred_element_type=jnp.float32)
          + jnp.dot(neighbor, alwn_ref[...],
                    preferred_element_type=jnp.float32) + alb_ref[...])
    score_ref[...] = _leaky(sc)


def _edge_call(nbrx, srcx, bond, encw, encb, rmat, smat, attw, attb,
               alwt, alwn, alb):
    const = lambda shape: pl.BlockSpec(shape, lambda i: (0, 0))
    return pl.pallas_call(
        _edge_body,
        grid=(_NEB,),
        in_specs=[
            pl.BlockSpec((_EBLK, _GD), lambda i: (i, 0)),
            pl.BlockSpec((_EBLK, _GD), lambda i: (i + _NEB, 0)),
            pl.BlockSpec((_EBLK, 10), lambda i: (i, 0)),
            const((10, _FP * _FP)),
            const((1, _FP * _FP)),
            const((_GD, _FP * _FP)),
            const((_FP * _FP, _FP)),
            const((_FP, _FP)),
            const((1, _FP)),
            const((_GD, 1)),
            const((_FP, 1)),
            const((1, 1)),
        ],
        out_specs=[
            pl.BlockSpec((_EBLK, _FP), lambda i: (i, 0)),
            pl.BlockSpec((_EBLK, 1), lambda i: (i, 0)),
        ],
        out_shape=[
            jax.ShapeDtypeStruct((_NBOND, _FP), jnp.float32),
            jax.ShapeDtypeStruct((_NBOND, 1), jnp.float32),
        ],
    )(nbrx, srcx, bond, encw, encb, rmat, smat, attw, attb, alwt, alwn, alb)


def _gru_body(num_ref, den_ref, h_ref, wih_ref, bih_ref, whh_ref, bhh_ref,
              out_ref):
    ctx = _elu(num_ref[...] / (den_ref[...] + 1e-8))
    hf = h_ref[...]
    h = hf[:, :_FP]
    gi = jnp.dot(ctx, wih_ref[...], preferred_element_type=jnp.float32) \
        + bih_ref[...]
    gh = jnp.dot(hf, whh_ref[...], preferred_element_type=jnp.float32) \
        + bhh_ref[...]
    r = _sigmoid(gi[:, :_FP] + gh[:, :_FP])
    z = _sigmoid(gi[:, _FP:2 * _FP] + gh[:, _FP:2 * _FP])
    n = jnp.tanh(gi[:, 2 * _FP:] + r * gh[:, 2 * _FP:])
    res = (1.0 - z) * n + z * h
    ow = out_ref.shape[1]
    if ow > _FP:
        res = jnp.concatenate(
            [res, jnp.zeros((res.shape[0], ow - _FP), jnp.float32)], axis=1)
    out_ref[...] = res


def _gru_call(num, den, h, wih, bih, whh, bhh, out_width=_FP):
    m = num.shape[0]
    hw = h.shape[1]
    full = lambda shape: pl.BlockSpec(shape, lambda: (0, 0))
    return pl.pallas_call(
        _gru_body,
        in_specs=[
            full((m, _FP)), full((m, 1)), full((m, hw)),
            full((_FP, 3 * _FP)), full((1, 3 * _FP)),
            full((hw, 3 * _FP)), full((1, 3 * _FP)),
        ],
        out_specs=full((m, out_width)),
        out_shape=jax.ShapeDtypeStruct((m, out_width), jnp.float32),
    )(num, den, h, wih, bih, whh, bhh)


def _pre_body(a_ref, w_ref, b_ref, o_ref):
    res = jnp.maximum(
        jnp.dot(a_ref[...], w_ref[...], preferred_element_type=jnp.float32)
        + b_ref[...], 0.0)
    o_ref[...] = jnp.concatenate(
        [res, jnp.zeros((res.shape[0], _GD - _FP), jnp.float32)], axis=1)


def _pre_call(a, w, b):
    m, k = a.shape
    n = w.shape[1]
    full = lambda shape: pl.BlockSpec(shape, lambda: (0, 0))
    return pl.pallas_call(
        _pre_body,
        in_specs=[full((m, k)), full((k, n)), full((1, n))],
        out_specs=full((m, _GD)),
        out_shape=jax.ShapeDtypeStruct((m, _GD), jnp.float32),
    )(a, w, b)


def _fold_lin(W, b, g, be):
    sg = _S * g
    return W.T * sg[None, :], (b * sg + be)[None, :]


def kernel(atom, bond, bond_index, mol_index, params):
    p = params
    src = bond_index[:, 0]
    nbr = bond_index[:, 1]
    idx_all = jnp.concatenate([nbr, src])

    eye = jnp.eye(_FP, dtype=jnp.float32)
    zpad = jnp.zeros((_GD - _FP, _FP), jnp.float32)
    rmat = jnp.concatenate(
        [jnp.repeat(eye, _FP, axis=1),
         jnp.zeros((_GD - _FP, _FP * _FP), jnp.float32)], axis=0)
    smat = jnp.tile(eye, (_FP, 1))

    prew, preb = _fold_lin(p['pre_W'], p['pre_b'], p['pre_g'], p['pre_be'])
    x = _pre_call(atom, prew, preb)

    for k in range(_K):
        encw, encb = _fold_lin(p['enc_W'][k], p['enc_b'][k], p['enc_g'][k],
                               p['enc_be'][k])
        attw, attb = _fold_lin(p['att_W'][k], p['att_b'][k], p['att_g'][k],
                               p['att_be'][k])
        alw = p['align_W'][k][0]
        alwt = jnp.concatenate([alw[:_FP, None], zpad[:, :1]], axis=0)
        alwn = alw[_FP:, None]
        alb = p['align_b'][k][None, :]

        xg = _sc_gather(x, idx_all)
        att_e, score = _edge_call(xg, xg, bond, encw, encb, rmat, smat,
                                  attw, attb, alwt, alwn, alb)
        m = jax.ops.segment_max(score, src, num_segments=_NATOM)
        e = jnp.exp(score - jnp.take(m, src, axis=0))
        seg = jax.ops.segment_sum(
            jnp.concatenate([e * att_e, e], axis=1), src,
            num_segments=_NATOM)
        x = _gru_call(seg[:, :_FP], seg[:, _FP:_FP + 1], x,
                      p['gru_Wih'][k].T, p['gru_bih'][k][None, :],
                      jnp.concatenate([p['gru_Whh'][k].T,
                                       jnp.zeros((_GD - _FP, 3 * _FP),
                                                 jnp.float32)], axis=0),
                      p['gru_bhh'][k][None, :], out_width=_GD)

    x = x[:, :_FP]
    superatom = jax.ops.segment_sum(x, mol_index, num_segments=_NMOL)
    for t in range(_T):
        se = jnp.take(superatom, mol_index, axis=0)
        alw = p['sg_align_W'][t][0]
        sc = _leaky(se @ alw[:_FP, None] + x @ alw[_FP:, None]
                    + p['sg_align_b'][t][None, :])
        m = jax.ops.segment_max(sc, mol_index, num_segments=_NMOL)
        e = jnp.exp(sc - jnp.take(m, mol_index, axis=0))
        attw, attb = _fold_lin(p['sg_att_W'][t], p['sg_att_b'][t],
                               p['sg_att_g'][t], p['sg_att_be'][t])
        att = x @ attw + attb
        seg = jax.ops.segment_sum(
            jnp.concatenate([e * att, e], axis=1), mol_index,
            num_segments=_NMOL)
        superatom = _gru_call(seg[:, :_FP], seg[:, _FP:_FP + 1], superatom,
                              p['sg_gru_Wih'][t].T,
                              p['sg_gru_bih'][t][None, :],
                              p['sg_gru_Whh'][t].T,
                              p['sg_gru_bhh'][t][None, :])

    predw, predb = _fold_lin(p['pred_W1'], p['pred_b1'], p['pred_g'],
                             p['pred_be'])
    h = jnp.maximum(superatom @ predw + predb, 0.0)
    return h @ p['pred_W2'].T + p['pred_b2'][None, :]


# speedup vs baseline: 1.7208x; 1.7208x over previous
import functools
import math

import jax
import jax.numpy as jnp
from jax import lax
from jax.experimental import pallas as pl
from jax.experimental.pallas import tpu as pltpu
from jax.experimental.pallas import tpu_sc as plsc

_FP = 32
_NATOM = 10000
_NBOND = 160000
_NMOL = 512
_K = 3
_T = 3
_S = 1.0 / math.sqrt(1.0 + 1e-6)
_EBLK = 2000
_NEB = _NBOND // _EBLK


def _leaky(x):
    return jnp.where(x >= 0, x, 0.01 * x)


def _sigmoid(x):
    return 1.0 / (1.0 + jnp.exp(-x))


def _elu(x):
    return jnp.where(x > 0, x, jnp.exp(jnp.minimum(x, 0.0)) - 1.0)


_NW = 32
_GB = _NBOND * 2
_GPW = _GB // _NW
_GCH = 400
_GNC = _GPW // _GCH
_GD = 128


def _sc_gather_body(table, idx, out, idx_v, rows_v, sem):
    wid = lax.axis_index("s") * 2 + lax.axis_index("c")
    base = wid * _GPW
    for c in range(_GNC):
        off = base + c * _GCH
        pltpu.sync_copy(idx.at[pl.ds(off, _GCH)], idx_v)
        pltpu.async_copy(table.at[idx_v], rows_v, sem).wait()
        pltpu.sync_copy(rows_v, out.at[pl.ds(off, _GCH)])


_sc_gather = pl.kernel(
    _sc_gather_body,
    mesh=plsc.VectorSubcoreMesh(core_axis_name="c", subcore_axis_name="s"),
    out_type=jax.ShapeDtypeStruct((_GB, _GD), jnp.float32),
    scratch_types=[
        pltpu.VMEM((_GCH,), jnp.int32),
        pltpu.VMEM((_GCH, _GD), jnp.float32),
        pltpu.SemaphoreType.DMA,
    ],
)


def _exp_f32(u):
    u = jnp.maximum(u, -87.0)
    y = u * 1.4426950408889634
    nf = (y + 12582912.0) - 12582912.0
    r = (u - nf * 0.6931471824645996) - nf * (-1.904654323148236e-09)
    p = 1.0 / 120.0
    p = p * r + 1.0 / 24.0
    p = p * r + 1.0 / 6.0
    p = p * r + 0.5
    p = p * r + 1.0
    p = p * r + 1.0
    ni = nf.astype(jnp.int32)
    scale = plsc.bitcast(
        jnp.left_shift(ni + 127, 23).astype(jnp.int32), jnp.float32)
    return p * scale


_SCH = 256
_SNCH = _NBOND // _SCH
_AROW = _GD
_ANR = 10240


def _sc_scatter_body(eout, m, src, z, out, m_v, idx_v, ein, acc):
    cid = lax.axis_index("c")
    sid = lax.axis_index("s")
    wid = sid * 2 + cid
    lanes = lax.iota(jnp.int32, 16)

    pltpu.sync_copy(m, m_v)

    zbase = sid * (_ANR // 16)
    pltpu.sync_copy(z.at[pl.ds(0, _ANR // 16)],
                    acc.at[pl.ds(zbase, _ANR // 16)])
    plsc.subcore_barrier()

    def chunk(i, _):
        c = wid + i * _NW

        @pl.when(c < _SNCH)
        def _():
            off = c * _SCH
            pltpu.sync_copy(src.at[pl.ds(off, _SCH)], idx_v)
            pltpu.sync_copy(eout.at[pl.ds(off, _SCH)], ein)
            for g in range(_SCH // 16):
                rows = lanes + (g * 16)
                sidx = idx_v[pl.ds(g * 16, 16)]
                mg = plsc.load_gather(m_v, [sidx])
                scol = jnp.full((16,), _FP, jnp.int32)
                scg = plsc.load_gather(ein, [rows, scol])
                eg = _exp_f32(scg - mg)
                plsc.store_scatter(ein, [rows, scol], eg)
                for l in range(_FP):
                    col = jnp.full((16,), l, jnp.int32)
                    vals = plsc.load_gather(ein, [rows, col]) * eg
                    plsc.store_scatter(ein, [rows, col], vals)
                pltpu.sync_copy(ein.at[pl.ds(g * 16, 16)], acc.at[sidx],
                                add=True)
        return 0

    lax.fori_loop(0, (_SNCH + _NW - 1) // _NW, chunk, 0)
    plsc.subcore_barrier()

    @pl.when(sid == 0)
    def _():
        pltpu.sync_copy(acc, out.at[cid])


_sc_scatter = pl.kernel(
    _sc_scatter_body,
    mesh=plsc.VectorSubcoreMesh(core_axis_name="c", subcore_axis_name="s"),
    compiler_params=pltpu.CompilerParams(needs_layout_passes=False),
    out_type=jax.ShapeDtypeStruct((2, _ANR, _AROW), jnp.float32),
    scratch_types=[
        pltpu.VMEM((_NATOM,), jnp.float32),
        pltpu.VMEM((_SCH,), jnp.int32),
        pltpu.VMEM((_SCH, _GD), jnp.float32),
        pltpu.VMEM_SHARED((_ANR, _AROW), jnp.float32),
    ],
)


def _edge_body(nbrx_ref, srcx_ref, bond_ref, encw_ref, encb_ref, rmat_ref,
               smat_ref, attw_ref, attb_ref, alwt_ref, alwn_ref, alb_ref,
               att_ref, score_ref):
    benc = jnp.dot(bond_ref[...], encw_ref[...],
                   preferred_element_type=jnp.float32) + encb_ref[...]
    benc = jnp.maximum(benc, 0.0)
    ah = jnp.dot(nbrx_ref[...], rmat_ref[...],
                 preferred_element_type=jnp.float32)
    neighbor = jnp.dot(ah * benc, smat_ref[...],
                       preferred_element_type=jnp.float32)
    att = jnp.dot(neighbor, attw_ref[...],
                  preferred_element_type=jnp.float32) + attb_ref[...]
    sc = (jnp.dot(srcx_ref[...], alwt_ref[...],
                  preferred_element_type=jnp.float32)
          + jnp.dot(neighbor, alwn_ref[...],
                    preferred_element_type=jnp.float32) + alb_ref[...])
    sc = _leaky(sc)
    att_ref[...] = jnp.concatenate(
        [att, sc, jnp.zeros((att.shape[0], _GD - _FP - 1), jnp.float32)],
        axis=1)
    score_ref[...] = sc


def _edge_call(nbrx, srcx, bond, encw, encb, rmat, smat, attw, attb,
               alwt, alwn, alb):
    const = lambda shape: pl.BlockSpec(shape, lambda i: (0, 0))
    return pl.pallas_call(
        _edge_body,
        grid=(_NEB,),
        in_specs=[
            pl.BlockSpec((_EBLK, _GD), lambda i: (i, 0)),
            pl.BlockSpec((_EBLK, _GD), lambda i: (i + _NEB, 0)),
            pl.BlockSpec((_EBLK, 10), lambda i: (i, 0)),
            const((10, _FP * _FP)),
            const((1, _FP * _FP)),
            const((_GD, _FP * _FP)),
            const((_FP * _FP, _FP)),
            const((_FP, _FP)),
            const((1, _FP)),
            const((_GD, 1)),
            const((_FP, 1)),
            const((1, 1)),
        ],
        out_specs=[
            pl.BlockSpec((_EBLK, _GD), lambda i: (i, 0)),
            pl.BlockSpec((_EBLK, 1), lambda i: (i, 0)),
        ],
        out_shape=[
            jax.ShapeDtypeStruct((_NBOND, _GD), jnp.float32),
            jax.ShapeDtypeStruct((_NBOND, 1), jnp.float32),
        ],
    )(nbrx, srcx, bond, encw, encb, rmat, smat, attw, attb, alwt, alwn, alb)


def _gru_core(ctx, hf, wih_ref, bih_ref, whh_ref, bhh_ref, out_ref):
    h = hf[:, :_FP]
    gi = jnp.dot(ctx, wih_ref[...], preferred_element_type=jnp.float32) \
        + bih_ref[...]
    gh = jnp.dot(hf, whh_ref[...], preferred_element_type=jnp.float32) \
        + bhh_ref[...]
    r = _sigmoid(gi[:, :_FP] + gh[:, :_FP])
    z = _sigmoid(gi[:, _FP:2 * _FP] + gh[:, _FP:2 * _FP])
    n = jnp.tanh(gi[:, 2 * _FP:] + r * gh[:, 2 * _FP:])
    res = (1.0 - z) * n + z * h
    ow = out_ref.shape[1]
    if ow > _FP:
        res = jnp.concatenate(
            [res, jnp.zeros((res.shape[0], ow - _FP), jnp.float32)], axis=1)
    out_ref[...] = res


def _gru_body(num_ref, den_ref, h_ref, wih_ref, bih_ref, whh_ref, bhh_ref,
              out_ref):
    ctx = _elu(num_ref[...] / (den_ref[...] + 1e-8))
    _gru_core(ctx, h_ref[...], wih_ref, bih_ref, whh_ref, bhh_ref, out_ref)


def _gru2_body(p0_ref, p1_ref, h_ref, wih_ref, bih_ref, whh_ref, bhh_ref,
               out_ref):
    s = p0_ref[...] + p1_ref[...]
    s = s[:_NATOM]
    ctx = _elu(s[:, :_FP] / (s[:, _FP:_FP + 1] + 1e-8))
    _gru_core(ctx, h_ref[...], wih_ref, bih_ref, whh_ref, bhh_ref, out_ref)


def _gru2_call(parts, h, wih, bih, whh, bhh, out_width):
    m = _NATOM
    hw = h.shape[1]
    full = lambda shape: pl.BlockSpec(shape, lambda: (0, 0))
    return pl.pallas_call(
        _gru2_body,
        in_specs=[
            full((_ANR, _AROW)), full((_ANR, _AROW)), full((m, hw)),
            full((_FP, 3 * _FP)), full((1, 3 * _FP)),
            full((hw, 3 * _FP)), full((1, 3 * _FP)),
        ],
        out_specs=full((m, out_width)),
        out_shape=jax.ShapeDtypeStruct((m, out_width), jnp.float32),
    )(parts[0], parts[1], h, wih, bih, whh, bhh)


def _gru_call(num, den, h, wih, bih, whh, bhh, out_width=_FP):
    m = num.shape[0]
    hw = h.shape[1]
    full = lambda shape: pl.BlockSpec(shape, lambda: (0, 0))
    return pl.pallas_call(
        _gru_body,
        in_specs=[
            full((m, _FP)), full((m, 1)), full((m, hw)),
            full((_FP, 3 * _FP)), full((1, 3 * _FP)),
            full((hw, 3 * _FP)), full((1, 3 * _FP)),
        ],
        out_specs=full((m, out_width)),
        out_shape=jax.ShapeDtypeStruct((m, out_width), jnp.float32),
    )(num, den, h, wih, bih, whh, bhh)


def _pre_body(a_ref, w_ref, b_ref, o_ref):
    res = jnp.maximum(
        jnp.dot(a_ref[...], w_ref[...], preferred_element_type=jnp.float32)
        + b_ref[...], 0.0)
    o_ref[...] = jnp.concatenate(
        [res, jnp.zeros((res.shape[0], _GD - _FP), jnp.float32)], axis=1)


def _pre_call(a, w, b):
    m, k = a.shape
    n = w.shape[1]
    full = lambda shape: pl.BlockSpec(shape, lambda: (0, 0))
    return pl.pallas_call(
        _pre_body,
        in_specs=[full((m, k)), full((k, n)), full((1, n))],
        out_specs=full((m, _GD)),
        out_shape=jax.ShapeDtypeStruct((m, _GD), jnp.float32),
    )(a, w, b)


def _fold_lin(W, b, g, be):
    sg = _S * g
    return W.T * sg[None, :], (b * sg + be)[None, :]


def kernel(atom, bond, bond_index, mol_index, params):
    p = params
    src = bond_index[:, 0]
    nbr = bond_index[:, 1]
    idx_all = jnp.concatenate([nbr, src])

    eye = jnp.eye(_FP, dtype=jnp.float32)
    zpad = jnp.zeros((_GD - _FP, _FP), jnp.float32)
    rmat = jnp.concatenate(
        [jnp.repeat(eye, _FP, axis=1),
         jnp.zeros((_GD - _FP, _FP * _FP), jnp.float32)], axis=0)
    smat = jnp.tile(eye, (_FP, 1))

    prew, preb = _fold_lin(p['pre_W'], p['pre_b'], p['pre_g'], p['pre_be'])
    x = _pre_call(atom, prew, preb)
    zacc = jnp.zeros((_ANR, _AROW), jnp.float32)

    for k in range(_K):
        encw, encb = _fold_lin(p['enc_W'][k], p['enc_b'][k], p['enc_g'][k],
                               p['enc_be'][k])
        attw, attb = _fold_lin(p['att_W'][k], p['att_b'][k], p['att_g'][k],
                               p['att_be'][k])
        alw = p['align_W'][k][0]
        alwt = jnp.concatenate([alw[:_FP, None], zpad[:, :1]], axis=0)
        alwn = alw[_FP:, None]
        alb = p['align_b'][k][None, :]

        xg = _sc_gather(x, idx_all)
        eout, score = _edge_call(xg, xg, bond, encw, encb, rmat, smat,
                                 attw, attb, alwt, alwn, alb)
        m = jax.ops.segment_max(score, src, num_segments=_NATOM)
        parts = _sc_scatter(eout, m.reshape(-1), src, zacc)
        x = _gru2_call(parts, x,
                       p['gru_Wih'][k].T, p['gru_bih'][k][None, :],
                       jnp.concatenate([p['gru_Whh'][k].T,
                                        jnp.zeros((_GD - _FP, 3 * _FP),
                                                  jnp.float32)], axis=0),
                       p['gru_bhh'][k][None, :], out_width=_GD)

    x = x[:, :_FP]
    superatom = jax.ops.segment_sum(x, mol_index, num_segments=_NMOL)
    for t in range(_T):
        se = jnp.take(superatom, mol_index, axis=0)
        alw = p['sg_align_W'][t][0]
        sc = _leaky(se @ alw[:_FP, None] + x @ alw[_FP:, None]
                    + p['sg_align_b'][t][None, :])
        m = jax.ops.segment_max(sc, mol_index, num_segments=_NMOL)
        e = jnp.exp(sc - jnp.take(m, mol_index, axis=0))
        attw, attb = _fold_lin(p['sg_att_W'][t], p['sg_att_b'][t],
                               p['sg_att_g'][t], p['sg_att_be'][t])
        att = x @ attw + attb
        seg = jax.ops.segment_sum(
            jnp.concatenate([e * att, e], axis=1), mol_index,
            num_segments=_NMOL)
        superatom = _gru_call(seg[:, :_FP], seg[:, _FP:_FP + 1], superatom,
                              p['sg_gru_Wih'][t].T,
                              p['sg_gru_bih'][t][None, :],
                              p['sg_gru_Whh'][t].T,
                              p['sg_gru_bhh'][t][None, :])

    predw, predb = _fold_lin(p['pred_W1'], p['pred_b1'], p['pred_g'],
                             p['pred_be'])
    h = jnp.maximum(superatom @ predw + predb, 0.0)
    return h @ p['pred_W2'].T + p['pred_b2'][None, :]
